# baseline (speedup 1.0000x reference)
import jax
import jax.numpy as jnp
from jax.experimental import pallas as pl
from jax.experimental.pallas import tpu as pltpu

N = 8192
TI = 2048
NI = N // TI


NR = 2
RC = TI // NR


def _chamfer_kernel(a_ref, bt_ref, r2_ref, q2_ref, out_ref, colmin_ref, rowsum_ref):
    i = pl.program_id(1)
    bt = bt_ref[0]
    q2 = q2_ref[0]
    part = 0.0
    col_mins = []
    for c in range(NR):
        rs = pl.ds(c * RC, RC)
        cross2 = jax.lax.dot_general(
            a_ref[0, rs], bt, (((1,), (0,)), ((), ())),
            preferred_element_type=jnp.float32,
        )
        d = (r2_ref[0, rs] + q2) + cross2
        row_min = jnp.maximum(jnp.min(d, axis=1), 0.0)
        part = part + jnp.sum(row_min)
        col_mins.append(jnp.min(d, axis=0, keepdims=True))
    col_min = jnp.minimum(col_mins[0], col_mins[1])

    @pl.when(i == 0)
    def _init():
        rowsum_ref[0, 0] = part
        colmin_ref[...] = col_min

    @pl.when(i != 0)
    def _acc():
        rowsum_ref[0, 0] = rowsum_ref[0, 0] + part
        colmin_ref[...] = jnp.minimum(colmin_ref[...], col_min)

    @pl.when(i == NI - 1)
    def _fin():
        colsum = jnp.sum(jnp.maximum(colmin_ref[...], 0.0))
        out_ref[0, 0, 0] = (rowsum_ref[0, 0] + colsum) * (1.0 / N)


def _chamfer(a, bt, r2, q2):
    batch = a.shape[0]
    out = pl.pallas_call(
        _chamfer_kernel,
        grid=(batch, NI),
        in_specs=[
            pl.BlockSpec((1, TI, 8), lambda b, i: (b, i, 0)),
            pl.BlockSpec((1, 8, N), lambda b, i: (b, 0, 0)),
            pl.BlockSpec((1, TI, 1), lambda b, i: (b, i, 0)),
            pl.BlockSpec((1, 1, N), lambda b, i: (b, 0, 0)),
        ],
        out_specs=pl.BlockSpec((1, 1, 1), lambda b, i: (b, 0, 0),
                               memory_space=pltpu.SMEM),
        out_shape=jax.ShapeDtypeStruct((batch, 1, 1), jnp.float32),
        scratch_shapes=[
            pltpu.VMEM((1, N), jnp.float32),
            pltpu.SMEM((1, 1), jnp.float32),
        ],
        compiler_params=pltpu.CompilerParams(
            dimension_semantics=("parallel", "arbitrary"),
            vmem_limit_bytes=100 * 1024 * 1024,
            allow_input_fusion=[True, True, True, True],
        ),
    )(a, bt, r2, q2)
    return out[:, 0, 0]


def _prep_and_chamfer(pc1, pc2):
    pc1 = pc1.astype(jnp.float32)
    pc2 = pc2.astype(jnp.float32)
    batch, n, _ = pc1.shape
    r2 = jnp.sum(pc1 * pc1, axis=-1, keepdims=True)
    q2 = jnp.sum(pc2 * pc2, axis=-1)[:, None, :]
    zpad = jnp.zeros((batch, n, 5), jnp.bfloat16)
    a = jnp.concatenate([(-2.0 * pc1).astype(jnp.bfloat16), zpad], axis=-1)
    b = jnp.concatenate([pc2.astype(jnp.bfloat16), zpad], axis=-1)
    bt = jnp.swapaxes(b, 1, 2)
    return _chamfer(a, bt, r2, q2)


def kernel(pc1, pc2):
    return _prep_and_chamfer(pc1, pc2)


# speedup vs baseline: 1.0140x; 1.0140x over previous
import jax
import jax.numpy as jnp
from jax.experimental import pallas as pl
from jax.experimental.pallas import tpu as pltpu

N = 8192
TI = 2048
NI = N // TI


def _chamfer_kernel(a_ref, bt_ref, r2_ref, q2_ref, out_ref, colmin_ref, rowsum_ref):
    i = pl.program_id(1)
    cross2 = jax.lax.dot_general(
        a_ref[0], bt_ref[0], (((1,), (0,)), ((), ())),
        preferred_element_type=jnp.float32,
    )
    d = (r2_ref[0] + q2_ref[0]) + cross2
    row_min = jnp.maximum(jnp.min(d, axis=1), 0.0)
    part = jnp.sum(row_min)
    col_min = jnp.min(d, axis=0, keepdims=True)

    @pl.when(i == 0)
    def _init():
        rowsum_ref[0, 0] = part
        colmin_ref[...] = col_min

    @pl.when(i != 0)
    def _acc():
        rowsum_ref[0, 0] = rowsum_ref[0, 0] + part
        colmin_ref[...] = jnp.minimum(colmin_ref[...], col_min)

    @pl.when(i == NI - 1)
    def _fin():
        colsum = jnp.sum(jnp.maximum(colmin_ref[...], 0.0))
        out_ref[0, 0, 0] = (rowsum_ref[0, 0] + colsum) * (1.0 / N)


def _chamfer(a, bt, r2, q2):
    batch = a.shape[0]
    out = pl.pallas_call(
        _chamfer_kernel,
        grid=(batch, NI),
        in_specs=[
            pl.BlockSpec((1, TI, 8), lambda b, i: (b, i, 0)),
            pl.BlockSpec((1, 8, N), lambda b, i: (b, 0, 0)),
            pl.BlockSpec((1, TI, 1), lambda b, i: (b, i, 0)),
            pl.BlockSpec((1, 1, N), lambda b, i: (b, 0, 0)),
        ],
        out_specs=pl.BlockSpec((1, 1, 1), lambda b, i: (b, 0, 0),
                               memory_space=pltpu.SMEM),
        out_shape=jax.ShapeDtypeStruct((batch, 1, 1), jnp.float32),
        scratch_shapes=[
            pltpu.VMEM((1, N), jnp.float32),
            pltpu.SMEM((1, 1), jnp.float32),
        ],
        compiler_params=pltpu.CompilerParams(
            dimension_semantics=("parallel", "arbitrary"),
            vmem_limit_bytes=100 * 1024 * 1024,
            allow_input_fusion=[True, True, True, True],
        ),
    )(a, bt, r2, q2)
    return out[:, 0, 0]


def _prep_and_chamfer(pc1, pc2):
    pc1 = pc1.astype(jnp.float32)
    pc2 = pc2.astype(jnp.float32)
    batch, n, _ = pc1.shape
    r2 = jnp.sum(pc1 * pc1, axis=-1, keepdims=True)
    q2 = jnp.sum(pc2 * pc2, axis=-1)[:, None, :]
    zpad = jnp.zeros((batch, n, 5), jnp.bfloat16)
    a = jnp.concatenate([(-2.0 * pc1).astype(jnp.bfloat16), zpad], axis=-1)
    b = jnp.concatenate([pc2.astype(jnp.bfloat16), zpad], axis=-1)
    bt = jnp.swapaxes(b, 1, 2)
    return _chamfer(a, bt, r2, q2)


def kernel(pc1, pc2):
    return _prep_and_chamfer(pc1, pc2)
